# baseline (speedup 1.0000x reference)
import jax
import jax.numpy as jnp
from jax.experimental import pallas as pl
from jax.experimental.pallas import tpu as pltpu

_B, _S, _D = 16, 2048, 128
_H = _D
_CHUNK = 512
_IGNORE_KINDS = (0, 1)
_ZBIG = 1e9


def _gru_ln_kernel(keep_ref, xT_ref, wih_ref, whh_ref, brz_ref, bhhn_ref,
                   g_ref, beta_ref, outT_ref, gx_ref, hall_ref, h_ref):
    @pl.when(pl.program_id(0) == 0)
    def _init():
        h_ref[...] = jnp.zeros_like(h_ref)

    n_rows = _CHUNK * _B
    x_bs = xT_ref[...]
    keep_bs = keep_ref[...]
    x2d = jnp.swapaxes(x_bs, 0, 1).reshape(n_rows, _D)
    keep2d = jnp.swapaxes(keep_bs, 0, 1).reshape(n_rows, 1)

    gx = jnp.dot(x2d.astype(jnp.bfloat16),
                 wih_ref[...].astype(jnp.bfloat16),
                 preferred_element_type=jnp.float32)
    gx = gx + brz_ref[...]
    gx_ref[:, 0:_H] = gx[:, 0:_H]
    gx_ref[:, _H:2 * _H] = gx[:, _H:2 * _H] + _ZBIG * (1.0 - keep2d)
    gx_ref[:, 2 * _H:] = gx[:, 2 * _H:]

    whh = whh_ref[...].astype(jnp.bfloat16)
    bhhn = bhhn_ref[...]
    hb = _B // 2

    def half_step(gx_t, gh, h):
        r = jax.nn.sigmoid(gx_t[:, 0:_H] + gh[:, 0:_H])
        z = jax.nn.sigmoid(gx_t[:, _H:2 * _H] + gh[:, _H:2 * _H])
        n = jnp.tanh(gx_t[:, 2 * _H:] + r * (gh[:, 2 * _H:] + bhhn))
        return (1.0 - z) * n + z * h

    def _dot(h):
        return jnp.dot(h.astype(jnp.bfloat16), whh,
                       preferred_element_type=jnp.float32)

    def step(t, carry):
        h_a, h_b, gh_b = carry
        gh_a = _dot(h_a)
        h_b = half_step(gx_ref[pl.ds(t * _B + hb, hb), :], gh_b, h_b)
        hall_ref[pl.ds(t * _B + hb, hb), :] = h_b
        gh_b_next = _dot(h_b)
        h_a = half_step(gx_ref[pl.ds(t * _B, hb), :], gh_a, h_a)
        hall_ref[pl.ds(t * _B, hb), :] = h_a
        return (h_a, h_b, gh_b_next)

    h_a0 = h_ref[0:hb, :]
    h_b0 = h_ref[hb:, :]
    gh_b0 = _dot(h_b0)
    h_a, h_b, _ = jax.lax.fori_loop(
        0, _CHUNK, step, (h_a0, h_b0, gh_b0), unroll=32)
    h_ref[0:hb, :] = h_a
    h_ref[hb:, :] = h_b

    hall = hall_ref[...]
    mu = jnp.mean(hall, axis=-1, keepdims=True)
    var = jnp.mean((hall - mu) ** 2, axis=-1, keepdims=True)
    ln = (hall - mu) * jax.lax.rsqrt(var + 1e-5) * g_ref[...] + beta_ref[...]
    ln_bs = jnp.swapaxes(ln.reshape(_CHUNK, _B, _D), 0, 1)
    outT_ref[...] = jnp.where(keep_bs > 0.0, ln_bs, x_bs)


def kernel(token_seqs_embeddings, token_type_sequences, sequences_lengths,
           W_ih, W_hh, b_ih, b_hh, ln_gamma, ln_beta):
    del sequences_lengths
    x = token_seqs_embeddings
    b, s, d = x.shape

    keep = jnp.ones((b, s), dtype=bool)
    for kind in _IGNORE_KINDS:
        keep = jnp.logical_and(keep, token_type_sequences != kind)
    keepf = keep.astype(jnp.float32)[:, :, None]

    brz = b_ih + jnp.concatenate(
        [b_hh[0:_H], b_hh[_H:2 * _H], jnp.zeros((_H,), b_hh.dtype)])

    grid = (s // _CHUNK,)
    outT = pl.pallas_call(
        _gru_ln_kernel,
        grid=grid,
        in_specs=[
            pl.BlockSpec((b, _CHUNK, 1), lambda i: (0, i, 0)),
            pl.BlockSpec((b, _CHUNK, d), lambda i: (0, i, 0)),
            pl.BlockSpec((d, 3 * _H), lambda i: (0, 0)),
            pl.BlockSpec((d, 3 * _H), lambda i: (0, 0)),
            pl.BlockSpec((1, 3 * _H), lambda i: (0, 0)),
            pl.BlockSpec((1, _H), lambda i: (0, 0)),
            pl.BlockSpec((1, d), lambda i: (0, 0)),
            pl.BlockSpec((1, d), lambda i: (0, 0)),
        ],
        out_specs=pl.BlockSpec((b, _CHUNK, d), lambda i: (0, i, 0)),
        out_shape=jax.ShapeDtypeStruct((b, s, d), x.dtype),
        scratch_shapes=[
            pltpu.VMEM((_CHUNK * b, 3 * _H), jnp.float32),
            pltpu.VMEM((_CHUNK * b, d), jnp.float32),
            pltpu.VMEM((b, d), jnp.float32),
        ],
        compiler_params=pltpu.CompilerParams(
            dimension_semantics=("arbitrary",),
        ),
    )(keepf, x, W_ih, W_hh, brz.reshape(1, -1),
      b_hh[2 * _H:].reshape(1, -1), ln_gamma.reshape(1, -1),
      ln_beta.reshape(1, -1))

    return outT


# speedup vs baseline: 1.0290x; 1.0290x over previous
import jax
import jax.numpy as jnp
from jax.experimental import pallas as pl
from jax.experimental.pallas import tpu as pltpu

_B, _S, _D = 16, 2048, 128
_H = _D
_CHUNK = 512
_IGNORE_KINDS = (0, 1)
_ZBIG = 1e9


def _gru_ln_kernel(keep_ref, xT_ref, wih_ref, whh_ref, brz_ref, bhhn_ref,
                   g_ref, beta_ref, outT_ref, gx_ref, hall_ref, h_ref):
    @pl.when(pl.program_id(0) == 0)
    def _init():
        h_ref[...] = jnp.zeros_like(h_ref)

    n_rows = _CHUNK * _B
    x_bs = xT_ref[...]
    keep_bs = keep_ref[...]
    x2d = jnp.swapaxes(x_bs, 0, 1).reshape(n_rows, _D)
    keep2d = jnp.swapaxes(keep_bs, 0, 1).reshape(n_rows, 1)

    gx = jnp.dot(x2d.astype(jnp.bfloat16),
                 wih_ref[...].astype(jnp.bfloat16),
                 preferred_element_type=jnp.float32)
    gx = gx + brz_ref[...]
    gx_ref[:, 0:_H] = gx[:, 0:_H]
    gx_ref[:, _H:2 * _H] = gx[:, _H:2 * _H] + _ZBIG * (1.0 - keep2d)
    gx_ref[:, 2 * _H:] = gx[:, 2 * _H:]

    whh = whh_ref[...].astype(jnp.bfloat16)
    bhhn = bhhn_ref[...]
    hb = _B // 2

    def half_step(gx_t, gh, h):
        r = 0.5 + 0.5 * jnp.tanh(0.5 * (gx_t[:, 0:_H] + gh[:, 0:_H]))
        z = 0.5 + 0.5 * jnp.tanh(0.5 * (gx_t[:, _H:2 * _H]
                                        + gh[:, _H:2 * _H]))
        n = jnp.tanh(gx_t[:, 2 * _H:] + r * (gh[:, 2 * _H:] + bhhn))
        return (1.0 - z) * n + z * h

    def _dot(h):
        return jnp.dot(h.astype(jnp.bfloat16), whh,
                       preferred_element_type=jnp.float32)

    def step(t, carry):
        h_a, h_b, gh_b = carry
        gh_a = _dot(h_a)
        h_b = half_step(gx_ref[pl.ds(t * _B + hb, hb), :], gh_b, h_b)
        hall_ref[pl.ds(t * _B + hb, hb), :] = h_b
        gh_b_next = _dot(h_b)
        h_a = half_step(gx_ref[pl.ds(t * _B, hb), :], gh_a, h_a)
        hall_ref[pl.ds(t * _B, hb), :] = h_a
        return (h_a, h_b, gh_b_next)

    h_a0 = h_ref[0:hb, :]
    h_b0 = h_ref[hb:, :]
    gh_b0 = _dot(h_b0)
    h_a, h_b, _ = jax.lax.fori_loop(
        0, _CHUNK, step, (h_a0, h_b0, gh_b0), unroll=32)
    h_ref[0:hb, :] = h_a
    h_ref[hb:, :] = h_b

    hall = hall_ref[...]
    mu = jnp.mean(hall, axis=-1, keepdims=True)
    var = jnp.mean((hall - mu) ** 2, axis=-1, keepdims=True)
    ln = (hall - mu) * jax.lax.rsqrt(var + 1e-5) * g_ref[...] + beta_ref[...]
    ln_bs = jnp.swapaxes(ln.reshape(_CHUNK, _B, _D), 0, 1)
    outT_ref[...] = jnp.where(keep_bs > 0.0, ln_bs, x_bs)


def kernel(token_seqs_embeddings, token_type_sequences, sequences_lengths,
           W_ih, W_hh, b_ih, b_hh, ln_gamma, ln_beta):
    del sequences_lengths
    x = token_seqs_embeddings
    b, s, d = x.shape

    keep = jnp.ones((b, s), dtype=bool)
    for kind in _IGNORE_KINDS:
        keep = jnp.logical_and(keep, token_type_sequences != kind)
    keepf = keep.astype(jnp.float32)[:, :, None]

    brz = b_ih + jnp.concatenate(
        [b_hh[0:_H], b_hh[_H:2 * _H], jnp.zeros((_H,), b_hh.dtype)])

    grid = (s // _CHUNK,)
    outT = pl.pallas_call(
        _gru_ln_kernel,
        grid=grid,
        in_specs=[
            pl.BlockSpec((b, _CHUNK, 1), lambda i: (0, i, 0)),
            pl.BlockSpec((b, _CHUNK, d), lambda i: (0, i, 0)),
            pl.BlockSpec((d, 3 * _H), lambda i: (0, 0)),
            pl.BlockSpec((d, 3 * _H), lambda i: (0, 0)),
            pl.BlockSpec((1, 3 * _H), lambda i: (0, 0)),
            pl.BlockSpec((1, _H), lambda i: (0, 0)),
            pl.BlockSpec((1, d), lambda i: (0, 0)),
            pl.BlockSpec((1, d), lambda i: (0, 0)),
        ],
        out_specs=pl.BlockSpec((b, _CHUNK, d), lambda i: (0, i, 0)),
        out_shape=jax.ShapeDtypeStruct((b, s, d), x.dtype),
        scratch_shapes=[
            pltpu.VMEM((_CHUNK * b, 3 * _H), jnp.float32),
            pltpu.VMEM((_CHUNK * b, d), jnp.float32),
            pltpu.VMEM((b, d), jnp.float32),
        ],
        compiler_params=pltpu.CompilerParams(
            dimension_semantics=("arbitrary",),
        ),
    )(keepf, x, W_ih, W_hh, brz.reshape(1, -1),
      b_hh[2 * _H:].reshape(1, -1), ln_gamma.reshape(1, -1),
      ln_beta.reshape(1, -1))

    return outT


# speedup vs baseline: 1.0309x; 1.0019x over previous
import jax
import jax.numpy as jnp
from jax.experimental import pallas as pl
from jax.experimental.pallas import tpu as pltpu

_B, _S, _D = 16, 2048, 128
_H = _D
_CHUNK = 512
_IGNORE_KINDS = (0, 1)
_ZBIG = 1e9


def _gru_ln_kernel(keep_ref, xT_ref, wih_ref, whh_ref, brz_ref, bhhn_ref,
                   g_ref, beta_ref, outT_ref, gx_ref, hall_ref, h_ref):
    @pl.when(pl.program_id(0) == 0)
    def _init():
        h_ref[...] = jnp.zeros_like(h_ref)

    n_rows = _CHUNK * _B
    x_bs = xT_ref[...]
    keep_bs = keep_ref[...]
    x2d = jnp.swapaxes(x_bs, 0, 1).reshape(n_rows, _D)
    keep2d = jnp.swapaxes(keep_bs, 0, 1).reshape(n_rows, 1)

    gx = jnp.dot(x2d.astype(jnp.bfloat16),
                 wih_ref[...].astype(jnp.bfloat16),
                 preferred_element_type=jnp.float32)
    gx = gx + brz_ref[...]
    gx_ref[:, 0:_H] = gx[:, 0:_H]
    gx_ref[:, _H:2 * _H] = gx[:, _H:2 * _H] + _ZBIG * (1.0 - keep2d)
    gx_ref[:, 2 * _H:] = gx[:, 2 * _H:]

    whh = whh_ref[...].astype(jnp.bfloat16)
    bhhn = bhhn_ref[...]
    hb = _B // 2

    def half_step(gx_t, gh, h):
        r = 0.5 + 0.5 * jnp.tanh(gx_t[:, 0:_H] + gh[:, 0:_H])
        z = 0.5 + 0.5 * jnp.tanh(gx_t[:, _H:2 * _H] + gh[:, _H:2 * _H])
        n = jnp.tanh(gx_t[:, 2 * _H:] + r * (gh[:, 2 * _H:] + bhhn))
        return (1.0 - z) * n + z * h

    def _dot(h):
        return jnp.dot(h.astype(jnp.bfloat16), whh,
                       preferred_element_type=jnp.float32)

    def step(t, carry):
        h_a, h_b, gh_b = carry
        gh_a = _dot(h_a)
        h_b = half_step(gx_ref[pl.ds(t * _B + hb, hb), :], gh_b, h_b)
        hall_ref[pl.ds(t * _B + hb, hb), :] = h_b
        gh_b_next = _dot(h_b)
        h_a = half_step(gx_ref[pl.ds(t * _B, hb), :], gh_a, h_a)
        hall_ref[pl.ds(t * _B, hb), :] = h_a
        return (h_a, h_b, gh_b_next)

    h_a0 = h_ref[0:hb, :]
    h_b0 = h_ref[hb:, :]
    gh_b0 = _dot(h_b0)
    h_a, h_b, _ = jax.lax.fori_loop(
        0, _CHUNK, step, (h_a0, h_b0, gh_b0), unroll=32)
    h_ref[0:hb, :] = h_a
    h_ref[hb:, :] = h_b

    hall = hall_ref[...]
    mu = jnp.mean(hall, axis=-1, keepdims=True)
    var = jnp.mean((hall - mu) ** 2, axis=-1, keepdims=True)
    ln = (hall - mu) * jax.lax.rsqrt(var + 1e-5) * g_ref[...] + beta_ref[...]
    ln_bs = jnp.swapaxes(ln.reshape(_CHUNK, _B, _D), 0, 1)
    outT_ref[...] = jnp.where(keep_bs > 0.0, ln_bs, x_bs)


def kernel(token_seqs_embeddings, token_type_sequences, sequences_lengths,
           W_ih, W_hh, b_ih, b_hh, ln_gamma, ln_beta):
    del sequences_lengths
    x = token_seqs_embeddings
    b, s, d = x.shape

    keep = jnp.ones((b, s), dtype=bool)
    for kind in _IGNORE_KINDS:
        keep = jnp.logical_and(keep, token_type_sequences != kind)
    keepf = keep.astype(jnp.float32)[:, :, None]

    brz = b_ih + jnp.concatenate(
        [b_hh[0:_H], b_hh[_H:2 * _H], jnp.zeros((_H,), b_hh.dtype)])
    col_scale = jnp.concatenate(
        [jnp.full((2 * _H,), 0.5, jnp.float32),
         jnp.ones((_H,), jnp.float32)])
    brz = brz * col_scale
    W_ih = W_ih * col_scale[None, :]
    W_hh = W_hh * col_scale[None, :]

    grid = (s // _CHUNK,)
    outT = pl.pallas_call(
        _gru_ln_kernel,
        grid=grid,
        in_specs=[
            pl.BlockSpec((b, _CHUNK, 1), lambda i: (0, i, 0)),
            pl.BlockSpec((b, _CHUNK, d), lambda i: (0, i, 0)),
            pl.BlockSpec((d, 3 * _H), lambda i: (0, 0)),
            pl.BlockSpec((d, 3 * _H), lambda i: (0, 0)),
            pl.BlockSpec((1, 3 * _H), lambda i: (0, 0)),
            pl.BlockSpec((1, _H), lambda i: (0, 0)),
            pl.BlockSpec((1, d), lambda i: (0, 0)),
            pl.BlockSpec((1, d), lambda i: (0, 0)),
        ],
        out_specs=pl.BlockSpec((b, _CHUNK, d), lambda i: (0, i, 0)),
        out_shape=jax.ShapeDtypeStruct((b, s, d), x.dtype),
        scratch_shapes=[
            pltpu.VMEM((_CHUNK * b, 3 * _H), jnp.float32),
            pltpu.VMEM((_CHUNK * b, d), jnp.float32),
            pltpu.VMEM((b, d), jnp.float32),
        ],
        compiler_params=pltpu.CompilerParams(
            dimension_semantics=("arbitrary",),
        ),
    )(keepf, x, W_ih, W_hh, brz.reshape(1, -1),
      b_hh[2 * _H:].reshape(1, -1), ln_gamma.reshape(1, -1),
      ln_beta.reshape(1, -1))

    return outT


# speedup vs baseline: 1.0321x; 1.0012x over previous
import jax
import jax.numpy as jnp
from jax.experimental import pallas as pl
from jax.experimental.pallas import tpu as pltpu

_B, _S, _D = 16, 2048, 128
_H = _D
_CHUNK = 256
_IGNORE_KINDS = (0, 1)
_ZBIG = 1e9


def _gru_ln_kernel(keep_ref, xT_ref, wih_ref, whh_ref, brz_ref, bhhn_ref,
                   g_ref, beta_ref, outT_ref, gx_ref, hall_ref, h_ref):
    @pl.when(pl.program_id(0) == 0)
    def _init():
        h_ref[...] = jnp.zeros_like(h_ref)

    n_rows = _CHUNK * _B
    x_bs = xT_ref[...]
    keep_bs = keep_ref[...]
    x2d = jnp.swapaxes(x_bs, 0, 1).reshape(n_rows, _D)
    keep2d = jnp.swapaxes(keep_bs, 0, 1).reshape(n_rows, 1)

    gx = jnp.dot(x2d.astype(jnp.bfloat16),
                 wih_ref[...].astype(jnp.bfloat16),
                 preferred_element_type=jnp.float32)
    gx = gx + brz_ref[...]
    gx_ref[:, 0:_H] = gx[:, 0:_H]
    gx_ref[:, _H:2 * _H] = gx[:, _H:2 * _H] + _ZBIG * (1.0 - keep2d)
    gx_ref[:, 2 * _H:] = gx[:, 2 * _H:]

    whh = whh_ref[...].astype(jnp.bfloat16)
    bhhn = bhhn_ref[...]
    hb = _B // 2

    def half_step(gx_t, gh, h):
        r = 0.5 + 0.5 * jnp.tanh(gx_t[:, 0:_H] + gh[:, 0:_H])
        z = 0.5 + 0.5 * jnp.tanh(gx_t[:, _H:2 * _H] + gh[:, _H:2 * _H])
        n = jnp.tanh(gx_t[:, 2 * _H:] + r * (gh[:, 2 * _H:] + bhhn))
        return (1.0 - z) * n + z * h

    def _dot(h):
        return jnp.dot(h.astype(jnp.bfloat16), whh,
                       preferred_element_type=jnp.float32)

    def step(t, carry):
        h_a, h_b, gh_b = carry
        gh_a = _dot(h_a)
        h_b = half_step(gx_ref[pl.ds(t * _B + hb, hb), :], gh_b, h_b)
        hall_ref[pl.ds(t * _B + hb, hb), :] = h_b
        gh_b_next = _dot(h_b)
        h_a = half_step(gx_ref[pl.ds(t * _B, hb), :], gh_a, h_a)
        hall_ref[pl.ds(t * _B, hb), :] = h_a
        return (h_a, h_b, gh_b_next)

    h_a0 = h_ref[0:hb, :]
    h_b0 = h_ref[hb:, :]
    gh_b0 = _dot(h_b0)
    h_a, h_b, _ = jax.lax.fori_loop(
        0, _CHUNK, step, (h_a0, h_b0, gh_b0), unroll=32)
    h_ref[0:hb, :] = h_a
    h_ref[hb:, :] = h_b

    hall = hall_ref[...]
    mu = jnp.mean(hall, axis=-1, keepdims=True)
    var = jnp.mean((hall - mu) ** 2, axis=-1, keepdims=True)
    ln = (hall - mu) * jax.lax.rsqrt(var + 1e-5) * g_ref[...] + beta_ref[...]
    ln_bs = jnp.swapaxes(ln.reshape(_CHUNK, _B, _D), 0, 1)
    outT_ref[...] = jnp.where(keep_bs > 0.0, ln_bs, x_bs)


def kernel(token_seqs_embeddings, token_type_sequences, sequences_lengths,
           W_ih, W_hh, b_ih, b_hh, ln_gamma, ln_beta):
    del sequences_lengths
    x = token_seqs_embeddings
    b, s, d = x.shape

    keep = jnp.ones((b, s), dtype=bool)
    for kind in _IGNORE_KINDS:
        keep = jnp.logical_and(keep, token_type_sequences != kind)
    keepf = keep.astype(jnp.float32)[:, :, None]

    brz = b_ih + jnp.concatenate(
        [b_hh[0:_H], b_hh[_H:2 * _H], jnp.zeros((_H,), b_hh.dtype)])
    col_scale = jnp.concatenate(
        [jnp.full((2 * _H,), 0.5, jnp.float32),
         jnp.ones((_H,), jnp.float32)])
    brz = brz * col_scale
    W_ih = W_ih * col_scale[None, :]
    W_hh = W_hh * col_scale[None, :]

    grid = (s // _CHUNK,)
    outT = pl.pallas_call(
        _gru_ln_kernel,
        grid=grid,
        in_specs=[
            pl.BlockSpec((b, _CHUNK, 1), lambda i: (0, i, 0)),
            pl.BlockSpec((b, _CHUNK, d), lambda i: (0, i, 0)),
            pl.BlockSpec((d, 3 * _H), lambda i: (0, 0)),
            pl.BlockSpec((d, 3 * _H), lambda i: (0, 0)),
            pl.BlockSpec((1, 3 * _H), lambda i: (0, 0)),
            pl.BlockSpec((1, _H), lambda i: (0, 0)),
            pl.BlockSpec((1, d), lambda i: (0, 0)),
            pl.BlockSpec((1, d), lambda i: (0, 0)),
        ],
        out_specs=pl.BlockSpec((b, _CHUNK, d), lambda i: (0, i, 0)),
        out_shape=jax.ShapeDtypeStruct((b, s, d), x.dtype),
        scratch_shapes=[
            pltpu.VMEM((_CHUNK * b, 3 * _H), jnp.float32),
            pltpu.VMEM((_CHUNK * b, d), jnp.float32),
            pltpu.VMEM((b, d), jnp.float32),
        ],
        compiler_params=pltpu.CompilerParams(
            dimension_semantics=("arbitrary",),
        ),
    )(keepf, x, W_ih, W_hh, brz.reshape(1, -1),
      b_hh[2 * _H:].reshape(1, -1), ln_gamma.reshape(1, -1),
      ln_beta.reshape(1, -1))

    return outT
